# baseline (speedup 1.0000x reference)
import jax
import jax.numpy as jnp
import numpy as np
from jax import lax
from jax.experimental import pallas as pl
from jax.experimental.pallas import tpu as pltpu
from jax.experimental.pallas import tpu_sc as plsc

ALPHA = 0.1
STEPS = 4

LANES = 16
NC, NS = 2, 16
NW = NC * NS
UNROLL = 14
NSEG = 4

_taps = np.array([ALPHA, 1.0 - 2.0 * ALPHA, ALPHA], dtype=np.float64)
_k = np.array([1.0])
for _ in range(STEPS):
    _k = np.convolve(_k, _taps)
D0, D1, D2, D3, D4 = (float(_k[STEPS + j]) for j in range(STEPS + 1))


def _edge_steps(rows):
    h = list(rows)
    for _ in range(STEPS):
        upd = [h[j] + ALPHA * (h[j + 1] - 2.0 * h[j] + h[j - 1])
               for j in range(1, 7)]
        h[1:7] = upd
    return h


def _sc_body(x_hbm, o_hbm, in_buf, out_h0, out_h1, si, so0, so1, so_e,
             *, B, L, D):
    dchunks = D // LANES
    strips_per_w = (B * dchunks) // NW
    wid = lax.axis_index("s") * NC + lax.axis_index("c")
    seg = (L - 8) // NSEG
    groups = seg // UNROLL
    halves = (out_h0, out_h1)
    sems = (so0, so1)

    def hbm_in(s):
        b, d0 = s // dchunks, (s % dchunks) * LANES
        return x_hbm.at[b, :, pl.ds(d0, LANES)]

    def hbm_out(s, r0, n):
        b, d0 = s // dchunks, (s % dchunks) * LANES
        return o_hbm.at[b, pl.ds(r0, n), pl.ds(d0, LANES)]

    first = wid * strips_per_w
    pltpu.async_copy(hbm_in(first), in_buf, si)

    def strip(k, carry):
        s = wid * strips_per_w + k
        pltpu.make_async_copy(hbm_in(s), in_buf, si).wait()

        w = tuple(in_buf[j] for j in range(8))
        h = _edge_steps(w)
        in_buf[1], in_buf[2], in_buf[3] = h[1], h[2], h[3]
        pltpu.async_copy(in_buf.at[pl.ds(0, 4)], hbm_out(s, 0, 4), so_e)

        for sg in range(NSEG):
            half, sem = halves[sg % 2], sems[sg % 2]
            if sg >= 2:
                pltpu.make_async_copy(half, hbm_out(s, 4 + sg * seg, seg),
                                      sem).wait()
            else:
                @pl.when(k > 0)
                def _():
                    pltpu.make_async_copy(half, hbm_out(s, 4 + sg * seg, seg),
                                          sem).wait()

            def group(t, w, sg=sg, half=half):
                base = 4 + sg * seg + t * UNROLL
                for u in range(UNROLL):
                    w8 = in_buf[base + u + 4]
                    out = (D0 * w[4] + D1 * (w[3] + w[5]) + D2 * (w[2] + w[6])
                           + D3 * (w[1] + w[7]) + D4 * (w[0] + w8))
                    half[t * UNROLL + u] = out
                    w = w[1:] + (w8,)
                return w

            w = lax.fori_loop(0, groups, group, w)
            pltpu.async_copy(half, hbm_out(s, 4 + sg * seg, seg), sem)

        t = _edge_steps(w)
        in_buf[L - 4], in_buf[L - 3], in_buf[L - 2] = t[4], t[5], t[6]
        pltpu.async_copy(in_buf.at[pl.ds(L - 4, 4)], hbm_out(s, L - 4, 4), so_e)
        pltpu.make_async_copy(in_buf.at[pl.ds(0, 4)], hbm_out(s, 0, 4), so_e).wait()
        pltpu.make_async_copy(in_buf.at[pl.ds(L - 4, 4)], hbm_out(s, L - 4, 4), so_e).wait()

        @pl.when(k + 1 < strips_per_w)
        def _():
            pltpu.async_copy(hbm_in(s + 1), in_buf, si)

        return carry

    lax.fori_loop(0, strips_per_w, strip, 0)
    last = wid * strips_per_w + strips_per_w - 1
    for sg in (NSEG - 2, NSEG - 1):
        pltpu.make_async_copy(halves[sg % 2], hbm_out(last, 4 + sg * seg, seg),
                              sems[sg % 2]).wait()


def _sc_mixer(x, sc_batches):
    B, L, D = x.shape
    assert D % LANES == 0 and (sc_batches * (D // LANES)) % NW == 0
    assert (L - 8) % NSEG == 0 and ((L - 8) // NSEG) % UNROLL == 0

    import functools
    body = functools.partial(_sc_body, B=sc_batches, L=L, D=D)
    mesh = plsc.VectorSubcoreMesh(core_axis_name="c", subcore_axis_name="s")
    return pl.kernel(
        body,
        out_type=jax.ShapeDtypeStruct((B, L, D), jnp.float32),
        mesh=mesh,
        scratch_types=[
            pltpu.VMEM((L, LANES), jnp.float32),
            pltpu.VMEM(((L - 8) // NSEG, LANES), jnp.float32),
            pltpu.VMEM(((L - 8) // NSEG, LANES), jnp.float32),
            pltpu.SemaphoreType.DMA,
            pltpu.SemaphoreType.DMA,
            pltpu.SemaphoreType.DMA,
            pltpu.SemaphoreType.DMA,
        ],
        compiler_params=pltpu.CompilerParams(use_tc_tiling_on_sc=False),
    )(x)


def _tc_block(x_ref, o_ref):
    y = x_ref[0]
    L = y.shape[0]

    def edges(h):
        for _ in range(STEPS):
            upd = h[1:7] + ALPHA * (h[2:8] - 2.0 * h[1:7] + h[0:6])
            h = jnp.concatenate([h[:1], upd, h[7:]], axis=0)
        return h

    h = edges(y[0:8])
    t = edges(y[L - 8:L])
    mid = (D0 * y[4:-4] + D1 * (y[3:-5] + y[5:-3]) + D2 * (y[2:-6] + y[6:-2])
           + D3 * (y[1:-7] + y[7:-1]) + D4 * (y[:-8] + y[8:]))
    o_ref[0] = jnp.concatenate(
        [y[:1], h[1:4], mid, t[4:7], y[-1:]], axis=0)


def _tc_fill(donor, x, sc_batches):
    B, L, D = x.shape
    W = 128

    def body(_, x_ref, o_ref):
        _tc_block(x_ref, o_ref)

    return pl.pallas_call(
        body,
        grid=(B - sc_batches, D // W),
        in_specs=[
            pl.BlockSpec((1, 8, W), lambda i, j: (0, 0, 0)),
            pl.BlockSpec((1, L, W), lambda i, j: (i + sc_batches, 0, j)),
        ],
        out_specs=pl.BlockSpec((1, L, W), lambda i, j: (i + sc_batches, 0, j)),
        out_shape=jax.ShapeDtypeStruct((B, L, D), jnp.float32),
        input_output_aliases={0: 0},
    )(donor, x)


SC_BATCHES = 1


@jax.jit
def kernel(x):
    sc_out = _sc_mixer(x, SC_BATCHES)
    return _tc_fill(sc_out, x, SC_BATCHES)


# speedup vs baseline: 1.8156x; 1.8156x over previous
import jax
import jax.numpy as jnp
import numpy as np
from jax import lax
from jax.experimental import pallas as pl
from jax.experimental.pallas import tpu as pltpu
from jax.experimental.pallas import tpu_sc as plsc

ALPHA = 0.1
STEPS = 4

LANES = 16
NC, NS = 2, 16
NW = NC * NS
UNROLL = 14
NSEG = 4

_taps = np.array([ALPHA, 1.0 - 2.0 * ALPHA, ALPHA], dtype=np.float64)
_k = np.array([1.0])
for _ in range(STEPS):
    _k = np.convolve(_k, _taps)
D0, D1, D2, D3, D4 = (float(_k[STEPS + j]) for j in range(STEPS + 1))


def _edge_steps(rows):
    h = list(rows)
    for _ in range(STEPS):
        upd = [h[j] + ALPHA * (h[j + 1] - 2.0 * h[j] + h[j - 1])
               for j in range(1, 7)]
        h[1:7] = upd
    return h


CH = 256
HALO_ROWS = 272
PAD = 8
WGROUP = 8


def _sc_body(x_hbm, o_hbm, in_buf, out_buf, si, so, *, B, L, D):
    dgroups = D // 128
    chunks = L // CH
    tasks_per_w = (B * dgroups * chunks) // NW
    wid = lax.axis_index("s") * NC + lax.axis_index("c")

    def task(k, carry):
        t = wid * tasks_per_w + k
        b = t // (dgroups * chunks)
        rem = t % (dgroups * chunks)
        dg, c = rem // chunks, rem % chunks
        start = pl.multiple_of(c * CH, 8)
        lo = pl.multiple_of(jnp.clip(start - 8, 0, L - HALO_ROWS), 8)
        lb = start - lo + PAD

        pltpu.async_copy(
            x_hbm.at[b, pl.ds(lo, HALO_ROWS), pl.ds(dg * 128, 128)],
            in_buf.at[pl.ds(PAD, HALO_ROWS)], si).wait()

        for cg in range(8):
            lane = pl.ds(cg * 16, LANES)
            w = tuple(in_buf[lb - 4 + j, lane] for j in range(8))

            def group(g, w, lane=lane):
                base = lb + g * WGROUP
                for u in range(WGROUP):
                    w8 = in_buf[base + u + 4, lane]
                    out = (D0 * w[4] + D1 * (w[3] + w[5]) + D2 * (w[2] + w[6])
                           + D3 * (w[1] + w[7]) + D4 * (w[0] + w8))
                    out_buf[g * WGROUP + u, lane] = out
                    w = w[1:] + (w8,)
                return w

            lax.fori_loop(0, CH // WGROUP, group, w)

        @pl.when(c == 0)
        def _():
            for cg in range(8):
                lane = pl.ds(cg * 16, LANES)
                h = _edge_steps(tuple(in_buf[PAD + j, lane] for j in range(8)))
                out_buf[0, lane] = in_buf[PAD, lane]
                out_buf[1, lane], out_buf[2, lane], out_buf[3, lane] = \
                    h[1], h[2], h[3]

        @pl.when(c == chunks - 1)
        def _():
            for cg in range(8):
                lane = pl.ds(cg * 16, LANES)
                base = PAD + HALO_ROWS - 8
                tl = _edge_steps(tuple(in_buf[base + j, lane] for j in range(8)))
                out_buf[CH - 4, lane], out_buf[CH - 3, lane], \
                    out_buf[CH - 2, lane] = tl[4], tl[5], tl[6]
                out_buf[CH - 1, lane] = in_buf[base + 7, lane]

        pltpu.async_copy(
            out_buf,
            o_hbm.at[b, pl.ds(start, CH), pl.ds(dg * 128, 128)], so).wait()
        return carry

    lax.fori_loop(0, tasks_per_w, task, 0)


def _sc_mixer(x, sc_batches):
    B, L, D = x.shape
    assert D % 128 == 0 and L % CH == 0
    assert (sc_batches * (D // 128) * (L // CH)) % NW == 0

    import functools
    body = functools.partial(_sc_body, B=sc_batches, L=L, D=D)
    mesh = plsc.VectorSubcoreMesh(core_axis_name="c", subcore_axis_name="s")
    return pl.kernel(
        body,
        out_type=jax.ShapeDtypeStruct((B, L, D), jnp.float32),
        mesh=mesh,
        scratch_types=[
            pltpu.VMEM((HALO_ROWS + 2 * PAD, 128), jnp.float32),
            pltpu.VMEM((CH, 128), jnp.float32),
            pltpu.SemaphoreType.DMA,
            pltpu.SemaphoreType.DMA,
        ],
    )(x)


def _tc_block(x_ref, o_ref):
    y = x_ref[0]
    L = y.shape[0]

    def edges(h):
        for _ in range(STEPS):
            upd = h[1:7] + ALPHA * (h[2:8] - 2.0 * h[1:7] + h[0:6])
            h = jnp.concatenate([h[:1], upd, h[7:]], axis=0)
        return h

    h = edges(y[0:8])
    t = edges(y[L - 8:L])
    mid = (D0 * y[4:-4] + D1 * (y[3:-5] + y[5:-3]) + D2 * (y[2:-6] + y[6:-2])
           + D3 * (y[1:-7] + y[7:-1]) + D4 * (y[:-8] + y[8:]))
    o_ref[0] = jnp.concatenate(
        [y[:1], h[1:4], mid, t[4:7], y[-1:]], axis=0)


def _tc_fill(donor, x, sc_batches):
    B, L, D = x.shape
    W = 128

    def body(_, x_ref, o_ref):
        _tc_block(x_ref, o_ref)

    return pl.pallas_call(
        body,
        grid=(B - sc_batches, D // W),
        in_specs=[
            pl.BlockSpec((1, 8, W), lambda i, j: (0, 0, 0)),
            pl.BlockSpec((1, L, W), lambda i, j: (i + sc_batches, 0, j)),
        ],
        out_specs=pl.BlockSpec((1, L, W), lambda i, j: (i + sc_batches, 0, j)),
        out_shape=jax.ShapeDtypeStruct((B, L, D), jnp.float32),
        input_output_aliases={0: 0},
    )(donor, x)


SC_BATCHES = 1


@jax.jit
def kernel(x):
    sc_out = _sc_mixer(x, SC_BATCHES)
    return _tc_fill(sc_out, x, SC_BATCHES)


# speedup vs baseline: 1.8816x; 1.0363x over previous
import jax
import jax.numpy as jnp
import numpy as np
from jax import lax
from jax.experimental import pallas as pl
from jax.experimental.pallas import tpu as pltpu
from jax.experimental.pallas import tpu_sc as plsc

ALPHA = 0.1
STEPS = 4

LANES = 16
NC, NS = 2, 16
NW = NC * NS
UNROLL = 14
NSEG = 4

_taps = np.array([ALPHA, 1.0 - 2.0 * ALPHA, ALPHA], dtype=np.float64)
_k = np.array([1.0])
for _ in range(STEPS):
    _k = np.convolve(_k, _taps)
D0, D1, D2, D3, D4 = (float(_k[STEPS + j]) for j in range(STEPS + 1))


def _edge_steps(rows):
    h = list(rows)
    for _ in range(STEPS):
        upd = [h[j] + ALPHA * (h[j + 1] - 2.0 * h[j] + h[j - 1])
               for j in range(1, 7)]
        h[1:7] = upd
    return h


CH = 128
HALO_ROWS = CH + 16
PAD = 8
WGROUP = 8


def _task_compute(in_buf, out_buf, lb, c, chunks):
    for cg in range(8):
        lane = pl.ds(cg * 16, LANES)
        w = tuple(in_buf[lb - 4 + j, lane] for j in range(8))

        def group(g, w, lane=lane):
            base = lb + g * WGROUP
            for u in range(WGROUP):
                w8 = in_buf[base + u + 4, lane]
                out = (D0 * w[4] + D1 * (w[3] + w[5]) + D2 * (w[2] + w[6])
                       + D3 * (w[1] + w[7]) + D4 * (w[0] + w8))
                out_buf[g * WGROUP + u, lane] = out
                w = w[1:] + (w8,)
            return w

        lax.fori_loop(0, CH // WGROUP, group, w)

    @pl.when(c == 0)
    def _():
        for cg in range(8):
            lane = pl.ds(cg * 16, LANES)
            h = _edge_steps(tuple(in_buf[PAD + j, lane] for j in range(8)))
            out_buf[0, lane] = in_buf[PAD, lane]
            out_buf[1, lane], out_buf[2, lane], out_buf[3, lane] = \
                h[1], h[2], h[3]

    @pl.when(c == chunks - 1)
    def _():
        for cg in range(8):
            lane = pl.ds(cg * 16, LANES)
            base = PAD + HALO_ROWS - 8
            tl = _edge_steps(tuple(in_buf[base + j, lane] for j in range(8)))
            out_buf[CH - 4, lane], out_buf[CH - 3, lane], \
                out_buf[CH - 2, lane] = tl[4], tl[5], tl[6]
            out_buf[CH - 1, lane] = in_buf[base + 7, lane]


def _sc_body(x_hbm, o_hbm, in_a, in_b, out_a, out_b, si_a, si_b, so_a, so_b,
             *, B, L, D):
    dgroups = D // 128
    chunks = L // CH
    tasks_per_w = (B * dgroups * chunks) // NW
    assert tasks_per_w % 2 == 0
    wid = lax.axis_index("s") * NC + lax.axis_index("c")
    first = wid * tasks_per_w
    ins, outs = (in_a, in_b), (out_a, out_b)
    sis, sos = (si_a, si_b), (so_a, so_b)

    def coords(t):
        b = t // (dgroups * chunks)
        rem = t % (dgroups * chunks)
        dg, c = rem // chunks, rem % chunks
        start = pl.multiple_of(c * CH, 8)
        lo = pl.multiple_of(jnp.clip(start - 8, 0, L - HALO_ROWS), 8)
        return b, dg, c, start, lo

    def in_copy(t, p):
        b, dg, c, start, lo = coords(t)
        return pltpu.make_async_copy(
            x_hbm.at[b, pl.ds(lo, HALO_ROWS), pl.ds(dg * 128, 128)],
            ins[p].at[pl.ds(PAD, HALO_ROWS)], sis[p])

    def out_copy(t, p):
        b, dg, c, start, lo = coords(t)
        return pltpu.make_async_copy(
            outs[p], o_hbm.at[b, pl.ds(start, CH), pl.ds(dg * 128, 128)],
            sos[p])

    in_copy(first, 0).start()

    def pair(pk, carry):
        t0 = first + 2 * pk
        for p, t in ((0, t0), (1, t0 + 1)):
            if p == 0:
                in_copy(t0 + 1, 1).start()
            else:
                @pl.when(pk + 1 < tasks_per_w // 2)
                def _():
                    in_copy(t0 + 2, 0).start()
            in_copy(t, p).wait()
            @pl.when(pk > 0)
            def _():
                out_copy(t, p).wait()
            b, dg, c, start, lo = coords(t)
            _task_compute(ins[p], outs[p], start - lo + PAD, c, chunks)
            out_copy(t, p).start()
        return carry

    lax.fori_loop(0, tasks_per_w // 2, pair, 0)
    for p in (0, 1):
        out_copy(first + tasks_per_w - 2 + p, p).wait()


def _sc_mixer(x, sc_batches):
    B, L, D = x.shape
    assert D % 128 == 0 and L % CH == 0
    assert (sc_batches * (D // 128) * (L // CH)) % NW == 0

    import functools
    body = functools.partial(_sc_body, B=sc_batches, L=L, D=D)
    mesh = plsc.VectorSubcoreMesh(core_axis_name="c", subcore_axis_name="s")
    return pl.kernel(
        body,
        out_type=jax.ShapeDtypeStruct((B, L, D), jnp.float32),
        mesh=mesh,
        scratch_types=[
            pltpu.VMEM((HALO_ROWS + 2 * PAD, 128), jnp.float32),
            pltpu.VMEM((HALO_ROWS + 2 * PAD, 128), jnp.float32),
            pltpu.VMEM((CH, 128), jnp.float32),
            pltpu.VMEM((CH, 128), jnp.float32),
            pltpu.SemaphoreType.DMA,
            pltpu.SemaphoreType.DMA,
            pltpu.SemaphoreType.DMA,
            pltpu.SemaphoreType.DMA,
        ],
    )(x)


def _tc_block(x_ref, o_ref):
    y = x_ref[0]
    L = y.shape[0]

    def edges(h):
        for _ in range(STEPS):
            upd = h[1:7] + ALPHA * (h[2:8] - 2.0 * h[1:7] + h[0:6])
            h = jnp.concatenate([h[:1], upd, h[7:]], axis=0)
        return h

    h = edges(y[0:8])
    t = edges(y[L - 8:L])
    mid = (D0 * y[4:-4] + D1 * (y[3:-5] + y[5:-3]) + D2 * (y[2:-6] + y[6:-2])
           + D3 * (y[1:-7] + y[7:-1]) + D4 * (y[:-8] + y[8:]))
    o_ref[0] = jnp.concatenate(
        [y[:1], h[1:4], mid, t[4:7], y[-1:]], axis=0)


def _tc_fill(donor, x, sc_batches):
    B, L, D = x.shape
    W = 128

    def body(_, x_ref, o_ref):
        _tc_block(x_ref, o_ref)

    return pl.pallas_call(
        body,
        grid=(B - sc_batches, D // W),
        in_specs=[
            pl.BlockSpec((1, 8, W), lambda i, j: (0, 0, 0)),
            pl.BlockSpec((1, L, W), lambda i, j: (i + sc_batches, 0, j)),
        ],
        out_specs=pl.BlockSpec((1, L, W), lambda i, j: (i + sc_batches, 0, j)),
        out_shape=jax.ShapeDtypeStruct((B, L, D), jnp.float32),
        input_output_aliases={0: 0},
    )(donor, x)


SC_BATCHES = 1


@jax.jit
def kernel(x):
    sc_out = _sc_mixer(x, SC_BATCHES)
    return _tc_fill(sc_out, x, SC_BATCHES)


# speedup vs baseline: 1.8820x; 1.0002x over previous
import jax
import jax.numpy as jnp
import numpy as np
from jax import lax
from jax.experimental import pallas as pl
from jax.experimental.pallas import tpu as pltpu
from jax.experimental.pallas import tpu_sc as plsc

ALPHA = 0.1
STEPS = 4

LANES = 16
NC, NS = 2, 16
NW = NC * NS
UNROLL = 14
NSEG = 4

_taps = np.array([ALPHA, 1.0 - 2.0 * ALPHA, ALPHA], dtype=np.float64)
_k = np.array([1.0])
for _ in range(STEPS):
    _k = np.convolve(_k, _taps)
D0, D1, D2, D3, D4 = (float(_k[STEPS + j]) for j in range(STEPS + 1))


def _edge_steps(rows):
    h = list(rows)
    for _ in range(STEPS):
        upd = [h[j] + ALPHA * (h[j + 1] - 2.0 * h[j] + h[j - 1])
               for j in range(1, 7)]
        h[1:7] = upd
    return h


CH = 128
HALO_ROWS = CH + 16
PAD = 8
WGROUP = 8


def _task_compute(in_buf, out_buf, lb, c, chunks):
    for cg in range(8):
        lane = pl.ds(cg * 16, LANES)
        w = tuple(in_buf[lb - 4 + j, lane] for j in range(8))

        def group(g, w, lane=lane):
            base = lb + g * WGROUP
            for u in range(WGROUP):
                w8 = in_buf[base + u + 4, lane]
                out = (D0 * w[4] + D1 * (w[3] + w[5]) + D2 * (w[2] + w[6])
                       + D3 * (w[1] + w[7]) + D4 * (w[0] + w8))
                out_buf[g * WGROUP + u, lane] = out
                w = w[1:] + (w8,)
            return w

        lax.fori_loop(0, CH // WGROUP, group, w)

    @pl.when(c == 0)
    def _():
        for cg in range(8):
            lane = pl.ds(cg * 16, LANES)
            h = _edge_steps(tuple(in_buf[PAD + j, lane] for j in range(8)))
            out_buf[0, lane] = in_buf[PAD, lane]
            out_buf[1, lane], out_buf[2, lane], out_buf[3, lane] = \
                h[1], h[2], h[3]

    @pl.when(c == chunks - 1)
    def _():
        for cg in range(8):
            lane = pl.ds(cg * 16, LANES)
            base = PAD + HALO_ROWS - 8
            tl = _edge_steps(tuple(in_buf[base + j, lane] for j in range(8)))
            out_buf[CH - 4, lane], out_buf[CH - 3, lane], \
                out_buf[CH - 2, lane] = tl[4], tl[5], tl[6]
            out_buf[CH - 1, lane] = in_buf[base + 7, lane]


def _sc_body(x_hbm, o_hbm, in_a, in_b, out_a, out_b, si_a, si_b, so_a, so_b,
             *, B, L, D):
    dgroups = D // 128
    chunks = L // CH
    tasks_per_w = (B * dgroups * chunks) // NW
    assert tasks_per_w % 2 == 0
    wid = lax.axis_index("s") * NC + lax.axis_index("c")
    first = wid * tasks_per_w
    ins, outs = (in_a, in_b), (out_a, out_b)
    sis, sos = (si_a, si_b), (so_a, so_b)

    def coords(t):
        b = t // (dgroups * chunks)
        rem = t % (dgroups * chunks)
        dg, c = rem // chunks, rem % chunks
        start = pl.multiple_of(c * CH, 8)
        lo = pl.multiple_of(jnp.clip(start - 8, 0, L - HALO_ROWS), 8)
        return b, dg, c, start, lo

    def in_copy(t, p):
        b, dg, c, start, lo = coords(t)
        return pltpu.make_async_copy(
            x_hbm.at[b, pl.ds(lo, HALO_ROWS), pl.ds(dg * 128, 128)],
            ins[p].at[pl.ds(PAD, HALO_ROWS)], sis[p])

    def out_copy(t, p):
        b, dg, c, start, lo = coords(t)
        return pltpu.make_async_copy(
            outs[p], o_hbm.at[b, pl.ds(start, CH), pl.ds(dg * 128, 128)],
            sos[p])

    in_copy(first, 0).start()

    def pair(pk, carry):
        t0 = first + 2 * pk
        for p, t in ((0, t0), (1, t0 + 1)):
            if p == 0:
                in_copy(t0 + 1, 1).start()
            else:
                @pl.when(pk + 1 < tasks_per_w // 2)
                def _():
                    in_copy(t0 + 2, 0).start()
            in_copy(t, p).wait()
            @pl.when(pk > 0)
            def _():
                out_copy(t, p).wait()
            b, dg, c, start, lo = coords(t)
            _task_compute(ins[p], outs[p], start - lo + PAD, c, chunks)
            out_copy(t, p).start()
        return carry

    lax.fori_loop(0, tasks_per_w // 2, pair, 0)
    for p in (0, 1):
        out_copy(first + tasks_per_w - 2 + p, p).wait()


def _sc_mixer(x, sc_batches):
    B, L, D = x.shape
    assert D % 128 == 0 and L % CH == 0
    assert (sc_batches * (D // 128) * (L // CH)) % NW == 0

    import functools
    body = functools.partial(_sc_body, B=sc_batches, L=L, D=D)
    mesh = plsc.VectorSubcoreMesh(core_axis_name="c", subcore_axis_name="s")
    return pl.kernel(
        body,
        out_type=jax.ShapeDtypeStruct((B, L, D), jnp.float32),
        mesh=mesh,
        scratch_types=[
            pltpu.VMEM((HALO_ROWS + 2 * PAD, 128), jnp.float32),
            pltpu.VMEM((HALO_ROWS + 2 * PAD, 128), jnp.float32),
            pltpu.VMEM((CH, 128), jnp.float32),
            pltpu.VMEM((CH, 128), jnp.float32),
            pltpu.SemaphoreType.DMA,
            pltpu.SemaphoreType.DMA,
            pltpu.SemaphoreType.DMA,
            pltpu.SemaphoreType.DMA,
        ],
    )(x)


def _tc_block(x_ref, o_ref):
    y = x_ref[0]
    L = y.shape[0]

    def edges(h):
        for _ in range(STEPS):
            upd = h[1:7] + ALPHA * (h[2:8] - 2.0 * h[1:7] + h[0:6])
            h = jnp.concatenate([h[:1], upd, h[7:]], axis=0)
        return h

    h = edges(y[0:8])
    t = edges(y[L - 8:L])
    mid = (D0 * y[4:-4] + D1 * (y[3:-5] + y[5:-3]) + D2 * (y[2:-6] + y[6:-2])
           + D3 * (y[1:-7] + y[7:-1]) + D4 * (y[:-8] + y[8:]))
    o_ref[0, 0:4] = jnp.concatenate([y[:1], h[1:4]], axis=0)
    o_ref[0, 4:L - 4] = mid
    o_ref[0, L - 4:L] = jnp.concatenate([t[4:7], y[-1:]], axis=0)


def _tc_fill(donor, x, sc_batches):
    B, L, D = x.shape
    W = 128

    def body(_, x_ref, o_ref):
        _tc_block(x_ref, o_ref)

    return pl.pallas_call(
        body,
        grid=(B - sc_batches, D // W),
        in_specs=[
            pl.BlockSpec((1, 8, W), lambda i, j: (0, 0, 0)),
            pl.BlockSpec((1, L, W), lambda i, j: (i + sc_batches, 0, j)),
        ],
        out_specs=pl.BlockSpec((1, L, W), lambda i, j: (i + sc_batches, 0, j)),
        out_shape=jax.ShapeDtypeStruct((B, L, D), jnp.float32),
        input_output_aliases={0: 0},
    )(donor, x)


SC_BATCHES = 1


@jax.jit
def kernel(x):
    sc_out = _sc_mixer(x, SC_BATCHES)
    return _tc_fill(sc_out, x, SC_BATCHES)


# speedup vs baseline: 2.0700x; 1.0999x over previous
import jax
import jax.numpy as jnp
import numpy as np
from jax import lax
from jax.experimental import pallas as pl
from jax.experimental.pallas import tpu as pltpu
from jax.experimental.pallas import tpu_sc as plsc

ALPHA = 0.1
STEPS = 4

LANES = 16
NC, NS = 2, 16
NW = NC * NS
UNROLL = 14
NSEG = 4

_taps = np.array([ALPHA, 1.0 - 2.0 * ALPHA, ALPHA], dtype=np.float64)
_k = np.array([1.0])
for _ in range(STEPS):
    _k = np.convolve(_k, _taps)
D0, D1, D2, D3, D4 = (float(_k[STEPS + j]) for j in range(STEPS + 1))


def _edge_steps(rows):
    h = list(rows)
    for _ in range(STEPS):
        upd = [h[j] + ALPHA * (h[j + 1] - 2.0 * h[j] + h[j - 1])
               for j in range(1, 7)]
        h[1:7] = upd
    return h


CH = 128
HALO_ROWS = CH + 16
PAD = 8
WGROUP = 8


def _task_compute(in_buf, out_buf, lb, c, chunks):
    for cg in range(8):
        lane = pl.ds(cg * 16, LANES)
        w = tuple(in_buf[lb - 4 + j, lane] for j in range(8))

        def group(g, w, lane=lane):
            base = lb + g * WGROUP
            for u in range(WGROUP):
                w8 = in_buf[base + u + 4, lane]
                out = (D0 * w[4] + D1 * (w[3] + w[5]) + D2 * (w[2] + w[6])
                       + D3 * (w[1] + w[7]) + D4 * (w[0] + w8))
                out_buf[g * WGROUP + u, lane] = out
                w = w[1:] + (w8,)
            return w

        lax.fori_loop(0, CH // WGROUP, group, w)

    @pl.when(c == 0)
    def _():
        for cg in range(8):
            lane = pl.ds(cg * 16, LANES)
            h = _edge_steps(tuple(in_buf[PAD + j, lane] for j in range(8)))
            out_buf[0, lane] = in_buf[PAD, lane]
            out_buf[1, lane], out_buf[2, lane], out_buf[3, lane] = \
                h[1], h[2], h[3]

    @pl.when(c == chunks - 1)
    def _():
        for cg in range(8):
            lane = pl.ds(cg * 16, LANES)
            base = PAD + HALO_ROWS - 8
            tl = _edge_steps(tuple(in_buf[base + j, lane] for j in range(8)))
            out_buf[CH - 4, lane], out_buf[CH - 3, lane], \
                out_buf[CH - 2, lane] = tl[4], tl[5], tl[6]
            out_buf[CH - 1, lane] = in_buf[base + 7, lane]


def _sc_body(x_hbm, o_hbm, in_a, in_b, out_a, out_b, si_a, si_b, so_a, so_b,
             *, B, L, D):
    dgroups = SC_DGROUPS
    chunks = L // CH
    tasks_per_w = (B * dgroups * chunks) // NW
    assert tasks_per_w % 2 == 0
    wid = lax.axis_index("s") * NC + lax.axis_index("c")
    first = wid * tasks_per_w
    ins, outs = (in_a, in_b), (out_a, out_b)
    sis, sos = (si_a, si_b), (so_a, so_b)

    def coords(t):
        b = t // (dgroups * chunks)
        rem = t % (dgroups * chunks)
        dg, c = rem // chunks, rem % chunks
        start = pl.multiple_of(c * CH, 8)
        lo = pl.multiple_of(jnp.clip(start - 8, 0, L - HALO_ROWS), 8)
        return b, dg, c, start, lo

    def in_copy(t, p):
        b, dg, c, start, lo = coords(t)
        return pltpu.make_async_copy(
            x_hbm.at[b, pl.ds(lo, HALO_ROWS), pl.ds(dg * 128, 128)],
            ins[p].at[pl.ds(PAD, HALO_ROWS)], sis[p])

    def out_copy(t, p):
        b, dg, c, start, lo = coords(t)
        return pltpu.make_async_copy(
            outs[p], o_hbm.at[b, pl.ds(start, CH), pl.ds(dg * 128, 128)],
            sos[p])

    in_copy(first, 0).start()

    def pair(pk, carry):
        t0 = first + 2 * pk
        for p, t in ((0, t0), (1, t0 + 1)):
            if p == 0:
                in_copy(t0 + 1, 1).start()
            else:
                @pl.when(pk + 1 < tasks_per_w // 2)
                def _():
                    in_copy(t0 + 2, 0).start()
            in_copy(t, p).wait()
            @pl.when(pk > 0)
            def _():
                out_copy(t, p).wait()
            b, dg, c, start, lo = coords(t)
            _task_compute(ins[p], outs[p], start - lo + PAD, c, chunks)
            out_copy(t, p).start()
        return carry

    lax.fori_loop(0, tasks_per_w // 2, pair, 0)
    for p in (0, 1):
        out_copy(first + tasks_per_w - 2 + p, p).wait()


def _sc_mixer(x, sc_batches):
    B, L, D = x.shape
    assert D % 128 == 0 and L % CH == 0
    assert (sc_batches * SC_DGROUPS * (L // CH)) % NW == 0

    import functools
    body = functools.partial(_sc_body, B=sc_batches, L=L, D=D)
    mesh = plsc.VectorSubcoreMesh(core_axis_name="c", subcore_axis_name="s")
    return pl.kernel(
        body,
        out_type=jax.ShapeDtypeStruct((B, L, D), jnp.float32),
        mesh=mesh,
        scratch_types=[
            pltpu.VMEM((HALO_ROWS + 2 * PAD, 128), jnp.float32),
            pltpu.VMEM((HALO_ROWS + 2 * PAD, 128), jnp.float32),
            pltpu.VMEM((CH, 128), jnp.float32),
            pltpu.VMEM((CH, 128), jnp.float32),
            pltpu.SemaphoreType.DMA,
            pltpu.SemaphoreType.DMA,
            pltpu.SemaphoreType.DMA,
            pltpu.SemaphoreType.DMA,
        ],
    )(x)


def _tc_block(x_ref, o_ref):
    y = x_ref[0]
    L = y.shape[0]

    def edges(h):
        for _ in range(STEPS):
            upd = h[1:7] + ALPHA * (h[2:8] - 2.0 * h[1:7] + h[0:6])
            h = jnp.concatenate([h[:1], upd, h[7:]], axis=0)
        return h

    h = edges(y[0:8])
    t = edges(y[L - 8:L])
    mid = (D0 * y[4:-4] + D1 * (y[3:-5] + y[5:-3]) + D2 * (y[2:-6] + y[6:-2])
           + D3 * (y[1:-7] + y[7:-1]) + D4 * (y[:-8] + y[8:]))
    o_ref[0, 0:4] = jnp.concatenate([y[:1], h[1:4]], axis=0)
    o_ref[0, 4:L - 4] = mid
    o_ref[0, L - 4:L] = jnp.concatenate([t[4:7], y[-1:]], axis=0)


def _tc_fill(donor, x, sc_batches):
    B, L, D = x.shape
    W = 128

    def body(_, x_ref, o_ref):
        _tc_block(x_ref, o_ref)

    return pl.pallas_call(
        body,
        grid=(B - sc_batches, D // W),
        in_specs=[
            pl.BlockSpec((1, 8, W), lambda i, j: (0, 0, 0)),
            pl.BlockSpec((1, L, W), lambda i, j: (i + sc_batches, 0, j)),
        ],
        out_specs=pl.BlockSpec((1, L, W), lambda i, j: (i + sc_batches, 0, j)),
        out_shape=jax.ShapeDtypeStruct((B, L, D), jnp.float32),
        input_output_aliases={0: 0},
    )(donor, x)


SC_BATCHES = 1
SC_DGROUPS = 4


def _tc_fill_rest(donor, x):
    B, L, D = x.shape
    W = 128

    def body(_, x_ref, o_ref):
        _tc_block(x_ref, o_ref)

    return pl.pallas_call(
        body,
        grid=(SC_BATCHES, D // W - SC_DGROUPS),
        in_specs=[
            pl.BlockSpec((1, 8, W), lambda i, j: (0, 0, 0)),
            pl.BlockSpec((1, L, W), lambda i, j: (i, 0, j + SC_DGROUPS)),
        ],
        out_specs=pl.BlockSpec((1, L, W), lambda i, j: (i, 0, j + SC_DGROUPS)),
        out_shape=jax.ShapeDtypeStruct((B, L, D), jnp.float32),
        input_output_aliases={0: 0},
    )(donor, x)


@jax.jit
def kernel(x):
    sc_out = _sc_mixer(x, SC_BATCHES)
    out = _tc_fill(sc_out, x, SC_BATCHES)
    return _tc_fill_rest(out, x)


# speedup vs baseline: 2.1939x; 1.0599x over previous
import jax
import jax.numpy as jnp
import numpy as np
from jax import lax
from jax.experimental import pallas as pl
from jax.experimental.pallas import tpu as pltpu
from jax.experimental.pallas import tpu_sc as plsc

ALPHA = 0.1
STEPS = 4

LANES = 16
NC, NS = 2, 16
NW = NC * NS
UNROLL = 14
NSEG = 4

_taps = np.array([ALPHA, 1.0 - 2.0 * ALPHA, ALPHA], dtype=np.float64)
_k = np.array([1.0])
for _ in range(STEPS):
    _k = np.convolve(_k, _taps)
D0, D1, D2, D3, D4 = (float(_k[STEPS + j]) for j in range(STEPS + 1))


def _edge_steps(rows):
    h = list(rows)
    for _ in range(STEPS):
        upd = [h[j] + ALPHA * (h[j + 1] - 2.0 * h[j] + h[j - 1])
               for j in range(1, 7)]
        h[1:7] = upd
    return h


CH = 128
HALO_ROWS = CH + 16
PAD = 8
WGROUP = 8


def _task_compute(in_buf, out_buf, lb, c, chunks):
    for cg in range(8):
        lane = pl.ds(cg * 16, LANES)
        w = tuple(in_buf[lb - 4 + j, lane] for j in range(8))

        def group(g, w, lane=lane):
            base = lb + g * WGROUP
            for u in range(WGROUP):
                w8 = in_buf[base + u + 4, lane]
                out = (D0 * w[4] + D1 * (w[3] + w[5]) + D2 * (w[2] + w[6])
                       + D3 * (w[1] + w[7]) + D4 * (w[0] + w8))
                out_buf[g * WGROUP + u, lane] = out
                w = w[1:] + (w8,)
            return w

        lax.fori_loop(0, CH // WGROUP, group, w)

    @pl.when(c == 0)
    def _():
        for cg in range(8):
            lane = pl.ds(cg * 16, LANES)
            h = _edge_steps(tuple(in_buf[PAD + j, lane] for j in range(8)))
            out_buf[0, lane] = in_buf[PAD, lane]
            out_buf[1, lane], out_buf[2, lane], out_buf[3, lane] = \
                h[1], h[2], h[3]

    @pl.when(c == chunks - 1)
    def _():
        for cg in range(8):
            lane = pl.ds(cg * 16, LANES)
            base = PAD + HALO_ROWS - 8
            tl = _edge_steps(tuple(in_buf[base + j, lane] for j in range(8)))
            out_buf[CH - 4, lane], out_buf[CH - 3, lane], \
                out_buf[CH - 2, lane] = tl[4], tl[5], tl[6]
            out_buf[CH - 1, lane] = in_buf[base + 7, lane]


def _sc_body(x_hbm, o_hbm, in_a, in_b, out_a, out_b, si_a, si_b, so_a, so_b,
             *, B, L, D):
    dgroups = SC_DGROUPS
    chunks = L // CH
    tasks_per_w = (B * dgroups * chunks) // NW
    assert tasks_per_w % 2 == 0
    wid = lax.axis_index("s") * NC + lax.axis_index("c")
    first = wid * tasks_per_w
    ins, outs = (in_a, in_b), (out_a, out_b)
    sis, sos = (si_a, si_b), (so_a, so_b)

    def coords(t):
        b = t // (dgroups * chunks)
        rem = t % (dgroups * chunks)
        dg, c = rem // chunks, rem % chunks
        start = pl.multiple_of(c * CH, 8)
        lo = pl.multiple_of(jnp.clip(start - 8, 0, L - HALO_ROWS), 8)
        return b, dg, c, start, lo

    def in_copy(t, p):
        b, dg, c, start, lo = coords(t)
        return pltpu.make_async_copy(
            x_hbm.at[b, pl.ds(lo, HALO_ROWS), pl.ds(dg * 128, 128)],
            ins[p].at[pl.ds(PAD, HALO_ROWS)], sis[p])

    def out_copy(t, p):
        b, dg, c, start, lo = coords(t)
        return pltpu.make_async_copy(
            outs[p], o_hbm.at[b, pl.ds(start, CH), pl.ds(dg * 128, 128)],
            sos[p])

    in_copy(first, 0).start()

    def pair(pk, carry):
        t0 = first + 2 * pk
        for p, t in ((0, t0), (1, t0 + 1)):
            if p == 0:
                in_copy(t0 + 1, 1).start()
            else:
                @pl.when(pk + 1 < tasks_per_w // 2)
                def _():
                    in_copy(t0 + 2, 0).start()
            in_copy(t, p).wait()
            @pl.when(pk > 0)
            def _():
                out_copy(t, p).wait()
            b, dg, c, start, lo = coords(t)
            _task_compute(ins[p], outs[p], start - lo + PAD, c, chunks)
            out_copy(t, p).start()
        return carry

    lax.fori_loop(0, tasks_per_w // 2, pair, 0)
    for p in (0, 1):
        out_copy(first + tasks_per_w - 2 + p, p).wait()


def _sc_mixer(x, sc_batches):
    B, L, D = x.shape
    assert D % 128 == 0 and L % CH == 0
    assert (sc_batches * SC_DGROUPS * (L // CH)) % NW == 0

    import functools
    body = functools.partial(_sc_body, B=sc_batches, L=L, D=D)
    mesh = plsc.VectorSubcoreMesh(core_axis_name="c", subcore_axis_name="s")
    return pl.kernel(
        body,
        out_type=jax.ShapeDtypeStruct((B, L, D), jnp.float32),
        mesh=mesh,
        scratch_types=[
            pltpu.VMEM((HALO_ROWS + 2 * PAD, 128), jnp.float32),
            pltpu.VMEM((HALO_ROWS + 2 * PAD, 128), jnp.float32),
            pltpu.VMEM((CH, 128), jnp.float32),
            pltpu.VMEM((CH, 128), jnp.float32),
            pltpu.SemaphoreType.DMA,
            pltpu.SemaphoreType.DMA,
            pltpu.SemaphoreType.DMA,
            pltpu.SemaphoreType.DMA,
        ],
    )(x)


def _tc_block(x_ref, o_ref):
    y = x_ref[0]
    L = y.shape[0]

    def edges(h):
        for _ in range(STEPS):
            upd = h[1:7] + ALPHA * (h[2:8] - 2.0 * h[1:7] + h[0:6])
            h = jnp.concatenate([h[:1], upd, h[7:]], axis=0)
        return h

    h = edges(y[0:8])
    t = edges(y[L - 8:L])
    mid = (D0 * y[4:-4] + D1 * (y[3:-5] + y[5:-3]) + D2 * (y[2:-6] + y[6:-2])
           + D3 * (y[1:-7] + y[7:-1]) + D4 * (y[:-8] + y[8:]))
    o_ref[0, 0:4] = jnp.concatenate([y[:1], h[1:4]], axis=0)
    o_ref[0, 4:L - 4] = mid
    o_ref[0, L - 4:L] = jnp.concatenate([t[4:7], y[-1:]], axis=0)


def _tc_fill(donor, x, sc_batches):
    B, L, D = x.shape
    W = 128

    def body(_, x_ref, o_ref):
        _tc_block(x_ref, o_ref)

    return pl.pallas_call(
        body,
        grid=(B - sc_batches, D // W),
        in_specs=[
            pl.BlockSpec((1, 8, W), lambda i, j: (0, 0, 0)),
            pl.BlockSpec((1, L, W), lambda i, j: (i + sc_batches, 0, j)),
        ],
        out_specs=pl.BlockSpec((1, L, W), lambda i, j: (i + sc_batches, 0, j)),
        out_shape=jax.ShapeDtypeStruct((B, L, D), jnp.float32),
        input_output_aliases={0: 0},
    )(donor, x)


SC_BATCHES = 1
SC_DGROUPS = 2


def _tc_fill_rest(donor, x):
    B, L, D = x.shape
    W = 128

    def body(_, x_ref, o_ref):
        _tc_block(x_ref, o_ref)

    return pl.pallas_call(
        body,
        grid=(SC_BATCHES, D // W - SC_DGROUPS),
        in_specs=[
            pl.BlockSpec((1, 8, W), lambda i, j: (0, 0, 0)),
            pl.BlockSpec((1, L, W), lambda i, j: (i, 0, j + SC_DGROUPS)),
        ],
        out_specs=pl.BlockSpec((1, L, W), lambda i, j: (i, 0, j + SC_DGROUPS)),
        out_shape=jax.ShapeDtypeStruct((B, L, D), jnp.float32),
        input_output_aliases={0: 0},
    )(donor, x)


@jax.jit
def kernel(x):
    sc_out = _sc_mixer(x, SC_BATCHES)
    out = _tc_fill(sc_out, x, SC_BATCHES)
    return _tc_fill_rest(out, x)
